# baseline (speedup 1.0000x reference)
import functools

import jax
import jax.numpy as jnp
import numpy as np
from jax.experimental import pallas as pl

_N = 10000
_E = 320000
_F = 128
_AVG_LOG = float(np.log(33.0))


def _mm_kernel(x_ref, w_ref, b_ref, o_ref):
    o_ref[...] = jnp.dot(x_ref[...], w_ref[...],
                         preferred_element_type=jnp.float32) + b_ref[...]


def _mm(x, w, b, block_rows):
    m, k = x.shape
    n = w.shape[1]
    assert m % block_rows == 0, (m, block_rows)
    return pl.pallas_call(
        _mm_kernel,
        grid=(m // block_rows,),
        in_specs=[
            pl.BlockSpec((block_rows, k), lambda i: (i, 0)),
            pl.BlockSpec((k, n), lambda i: (0, 0)),
            pl.BlockSpec((n,), lambda i: (0,)),
        ],
        out_specs=pl.BlockSpec((block_rows, n), lambda i: (i, 0)),
        out_shape=jax.ShapeDtypeStruct((m, n), jnp.float32),
    )(x, w, b)


def kernel(x, edge_index, edge_attr, pos_edge_index, pos_edge_attr,
           neg_edge_index, neg_edge_attr, node_W, node_b, edge_W, edge_b,
           preW, preb, postW, postb, linW, linb, bn_g, bn_b, e1W, e1b,
           e2W, e2b):
    F = _F
    src, dst = edge_index[0], edge_index[1]
    n = x.shape[0]

    x = _mm(x, node_W, node_b, 1000)
    ea = _mm(edge_attr, edge_W, edge_b, 2000)
    pea = _mm(pos_edge_attr, edge_W, edge_b, 1000)
    nea = _mm(neg_edge_attr, edge_W, edge_b, 1000)

    cnt = jax.ops.segment_sum(jnp.ones((_E,), jnp.float32), dst,
                              num_segments=n)
    denom = jnp.clip(cnt, 1.0)[:, None]
    has = (cnt > 0)[:, None]
    amp = jnp.log(denom + 1.0) / _AVG_LOG
    att = _AVG_LOG / jnp.log(denom + 1.0)
    zerob = jnp.zeros((F,), jnp.float32)

    for i in range(2):
        A = _mm(x, preW[i][:F], zerob, 1000)
        B = _mm(x, preW[i][F:2 * F], zerob, 1000)
        C = _mm(ea, preW[i][2 * F:], preb[i], 2000)
        m = A[dst] + B[src] + C
        s1 = jax.ops.segment_sum(m, dst, num_segments=n)
        s2 = jax.ops.segment_sum(m * m, dst, num_segments=n)
        mean = s1 / denom
        var = s2 / denom - mean ** 2
        std = jnp.sqrt(jnp.maximum(var, 0.0) + 1e-5)
        mn = jnp.where(has, jax.ops.segment_min(m, dst, num_segments=n), 0.0)
        mx = jnp.where(has, jax.ops.segment_max(m, dst, num_segments=n), 0.0)
        agg = jnp.concatenate([mean, mn, mx, std], axis=-1)
        out = jnp.concatenate([x, agg, agg * amp, agg * att], axis=-1)
        out = _mm(out, postW[i], postb[i], 1000)
        c = _mm(out, linW[i], linb[i], 1000)
        mu = c.mean(0)
        v = ((c - mu) ** 2).mean(0)
        cbn = (c - mu) / jnp.sqrt(v + 1e-5) * bn_g[i] + bn_b[i]
        x = (x + jax.nn.relu(cbn)) / 2.0
        S = _mm(x, e1W[i][:F], zerob, 1000)
        D = _mm(x, e1W[i][F:2 * F], zerob, 1000)
        G = _mm(ea, e1W[i][2 * F:], e1b[i], 2000)
        T = jax.nn.relu(S[src] + D[dst] + G)
        ea = ea + _mm(T, e2W[i], e2b[i], 2000) * 0.5

    return (x, pea, nea)


# speedup vs baseline: 1.0355x; 1.0355x over previous
import functools

import jax
import jax.numpy as jnp
import numpy as np
from jax.experimental import pallas as pl

_N = 10000
_E = 320000
_F = 128
_AVG_LOG = float(np.log(33.0))


def _mm_kernel(x_ref, w_ref, b_ref, o_ref):
    o_ref[...] = jnp.dot(x_ref[...], w_ref[...],
                         preferred_element_type=jnp.float32) + b_ref[...]


def _mm(x, w, b, block_rows):
    m, k = x.shape
    n = w.shape[1]
    assert m % block_rows == 0, (m, block_rows)
    return pl.pallas_call(
        _mm_kernel,
        grid=(m // block_rows,),
        in_specs=[
            pl.BlockSpec((block_rows, k), lambda i: (i, 0)),
            pl.BlockSpec((k, n), lambda i: (0, 0)),
            pl.BlockSpec((n,), lambda i: (0,)),
        ],
        out_specs=pl.BlockSpec((block_rows, n), lambda i: (i, 0)),
        out_shape=jax.ShapeDtypeStruct((m, n), jnp.float32),
    )(x, w, b)


def kernel(x, edge_index, edge_attr, pos_edge_index, pos_edge_attr,
           neg_edge_index, neg_edge_attr, node_W, node_b, edge_W, edge_b,
           preW, preb, postW, postb, linW, linb, bn_g, bn_b, e1W, e1b,
           e2W, e2b):
    F = _F
    src, dst = edge_index[0], edge_index[1]
    n = x.shape[0]

    perm = jnp.argsort(dst)
    src, dst = src[perm], dst[perm]
    edge_attr = edge_attr[perm]

    x = _mm(x, node_W, node_b, 1000)
    ea = _mm(edge_attr, edge_W, edge_b, 2000)
    pea = _mm(pos_edge_attr, edge_W, edge_b, 1000)
    nea = _mm(neg_edge_attr, edge_W, edge_b, 1000)

    cnt = jax.ops.segment_sum(jnp.ones((_E,), jnp.float32), dst,
                              num_segments=n, indices_are_sorted=True)
    denom = jnp.clip(cnt, 1.0)[:, None]
    has = (cnt > 0)[:, None]
    amp = jnp.log(denom + 1.0) / _AVG_LOG
    att = _AVG_LOG / jnp.log(denom + 1.0)
    zerob = jnp.zeros((F,), jnp.float32)

    for i in range(2):
        A = _mm(x, preW[i][:F], zerob, 1000)
        B = _mm(x, preW[i][F:2 * F], zerob, 1000)
        C = _mm(ea, preW[i][2 * F:], preb[i], 2000)
        m = A[dst] + B[src] + C
        s1 = jax.ops.segment_sum(m, dst, num_segments=n, indices_are_sorted=True)
        s2 = jax.ops.segment_sum(m * m, dst, num_segments=n, indices_are_sorted=True)
        mean = s1 / denom
        var = s2 / denom - mean ** 2
        std = jnp.sqrt(jnp.maximum(var, 0.0) + 1e-5)
        mn = jnp.where(has, jax.ops.segment_min(m, dst, num_segments=n, indices_are_sorted=True), 0.0)
        mx = jnp.where(has, jax.ops.segment_max(m, dst, num_segments=n, indices_are_sorted=True), 0.0)
        agg = jnp.concatenate([mean, mn, mx, std], axis=-1)
        out = jnp.concatenate([x, agg, agg * amp, agg * att], axis=-1)
        out = _mm(out, postW[i], postb[i], 1000)
        c = _mm(out, linW[i], linb[i], 1000)
        mu = c.mean(0)
        v = ((c - mu) ** 2).mean(0)
        cbn = (c - mu) / jnp.sqrt(v + 1e-5) * bn_g[i] + bn_b[i]
        x = (x + jax.nn.relu(cbn)) / 2.0
        S = _mm(x, e1W[i][:F], zerob, 1000)
        D = _mm(x, e1W[i][F:2 * F], zerob, 1000)
        G = _mm(ea, e1W[i][2 * F:], e1b[i], 2000)
        T = jax.nn.relu(S[src] + D[dst] + G)
        ea = ea + _mm(T, e2W[i], e2b[i], 2000) * 0.5

    return (x, pea, nea)


# speedup vs baseline: 1.1174x; 1.0791x over previous
import functools

import jax
import jax.numpy as jnp
import numpy as np
from jax import lax
from jax.experimental import pallas as pl
from jax.experimental.pallas import tpu as pltpu
from jax.experimental.pallas import tpu_sc as plsc

_N = 10000
_E = 320000
_F = 128
_AVG_LOG = float(np.log(33.0))

_NC = 2
_NS = 16
_NW = _NC * _NS
_EPT = _E // _NW
_CHM = 200


def _mm_kernel(x_ref, w_ref, b_ref, o_ref):
    o_ref[...] = jnp.dot(x_ref[...], w_ref[...],
                         preferred_element_type=jnp.float32) + b_ref[...]


def _mm(x, w, b, block_rows):
    m, k = x.shape
    n = w.shape[1]
    assert m % block_rows == 0, (m, block_rows)
    return pl.pallas_call(
        _mm_kernel,
        grid=(m // block_rows,),
        in_specs=[
            pl.BlockSpec((block_rows, k), lambda i: (i, 0)),
            pl.BlockSpec((k, n), lambda i: (0, 0)),
            pl.BlockSpec((n,), lambda i: (0,)),
        ],
        out_specs=pl.BlockSpec((block_rows, n), lambda i: (i, 0)),
        out_shape=jax.ShapeDtypeStruct((m, n), jnp.float32),
    )(x, w, b)


def _edge_map_body(relu, a_hbm, b_hbm, c_hbm, ai_hbm, bi_hbm, out_hbm,
                   ai_v, bi_v, a_rows, b_rows, c_rows, t_rows, sem1, sem2):
    wid = lax.axis_index("s") * _NC + lax.axis_index("c")
    base = wid * _EPT

    def chunk(k, carry):
        cb = base + k * _CHM
        pltpu.sync_copy(ai_hbm.at[pl.ds(cb, _CHM)], ai_v)
        pltpu.sync_copy(bi_hbm.at[pl.ds(cb, _CHM)], bi_v)
        pltpu.sync_copy(c_hbm.at[pl.ds(cb, _CHM)], c_rows)
        cp1 = pltpu.async_copy(a_hbm.at[ai_v], a_rows, sem1)
        cp2 = pltpu.async_copy(b_hbm.at[bi_v], b_rows, sem2)
        cp1.wait()
        cp2.wait()

        def edge(j, carry2):
            for g in range(_F // 16):
                sl = pl.ds(g * 16, 16)
                t = a_rows[j, sl] + b_rows[j, sl] + c_rows[j, sl]
                if relu:
                    t = jnp.maximum(t, 0.0)
                t_rows[j, sl] = t
            return carry2

        lax.fori_loop(0, _CHM, edge, 0, unroll=2)
        pltpu.sync_copy(t_rows, out_hbm.at[pl.ds(cb, _CHM)])
        return carry

    lax.fori_loop(0, _EPT // _CHM, chunk, 0)


def _edge_map(a, b, c, ai, bi, relu):
    mesh = plsc.VectorSubcoreMesh(core_axis_name="c", subcore_axis_name="s")
    return pl.kernel(
        functools.partial(_edge_map_body, relu),
        mesh=mesh,
        out_type=jax.ShapeDtypeStruct((_E, _F), jnp.float32),
        scratch_types=[
            pltpu.VMEM((_CHM,), jnp.int32),
            pltpu.VMEM((_CHM,), jnp.int32),
            pltpu.VMEM((_CHM, _F), jnp.float32),
            pltpu.VMEM((_CHM, _F), jnp.float32),
            pltpu.VMEM((_CHM, _F), jnp.float32),
            pltpu.VMEM((_CHM, _F), jnp.float32),
            pltpu.SemaphoreType.DMA,
            pltpu.SemaphoreType.DMA,
        ],
    )(a, b, c, ai, bi)


def kernel(x, edge_index, edge_attr, pos_edge_index, pos_edge_attr,
           neg_edge_index, neg_edge_attr, node_W, node_b, edge_W, edge_b,
           preW, preb, postW, postb, linW, linb, bn_g, bn_b, e1W, e1b,
           e2W, e2b):
    F = _F
    src, dst = edge_index[0], edge_index[1]
    n = x.shape[0]

    perm = jnp.argsort(dst)
    src, dst = src[perm], dst[perm]
    edge_attr = edge_attr[perm]
    offsets = jnp.searchsorted(dst, jnp.arange(n + 1, dtype=jnp.int32),
                               method='scan_unrolled').astype(jnp.int32)
    cnt = (offsets[1:] - offsets[:-1]).astype(jnp.float32)

    x = _mm(x, node_W, node_b, 1000)
    ea = _mm(edge_attr, edge_W, edge_b, 2000)
    pea = _mm(pos_edge_attr, edge_W, edge_b, 1000)
    nea = _mm(neg_edge_attr, edge_W, edge_b, 1000)

    denom = jnp.clip(cnt, 1.0)[:, None]
    has = (cnt > 0)[:, None]
    amp = jnp.log(denom + 1.0) / _AVG_LOG
    att = _AVG_LOG / jnp.log(denom + 1.0)
    zerob = jnp.zeros((F,), jnp.float32)

    for i in range(2):
        A = _mm(x, preW[i][:F], zerob, 1000)
        B = _mm(x, preW[i][F:2 * F], zerob, 1000)
        C = _mm(ea, preW[i][2 * F:], preb[i], 2000)
        m = _edge_map(A, B, C, dst, src, relu=False)
        s1 = jax.ops.segment_sum(m, dst, num_segments=n,
                                 indices_are_sorted=True)
        s2 = jax.ops.segment_sum(m * m, dst, num_segments=n,
                                 indices_are_sorted=True)
        mean = s1 / denom
        var = s2 / denom - mean ** 2
        std = jnp.sqrt(jnp.maximum(var, 0.0) + 1e-5)
        mn = jnp.where(has, jax.ops.segment_min(m, dst, num_segments=n,
                                                indices_are_sorted=True), 0.0)
        mx = jnp.where(has, jax.ops.segment_max(m, dst, num_segments=n,
                                                indices_are_sorted=True), 0.0)
        agg = jnp.concatenate([mean, mn, mx, std], axis=-1)
        P0 = _mm(agg, postW[i][F:F + 4 * F], zerob, 1000)
        P1 = _mm(agg, postW[i][F + 4 * F:F + 8 * F], zerob, 1000)
        P2 = _mm(agg, postW[i][F + 8 * F:], zerob, 1000)
        out = _mm(x, postW[i][:F], postb[i], 1000) + P0 + amp * P1 + att * P2
        c = _mm(out, linW[i], linb[i], 1000)
        mu = c.mean(0)
        v = ((c - mu) ** 2).mean(0)
        cbn = (c - mu) / jnp.sqrt(v + 1e-5) * bn_g[i] + bn_b[i]
        x = (x + jax.nn.relu(cbn)) / 2.0
        S = _mm(x, e1W[i][:F], zerob, 1000)
        D = _mm(x, e1W[i][F:2 * F], zerob, 1000)
        G = _mm(ea, e1W[i][2 * F:], e1b[i], 2000)
        T = _edge_map(S, D, G, src, dst, relu=True)
        ea = ea + _mm(T, e2W[i], e2b[i], 2000) * 0.5

    return (x, pea, nea)
